# Initial kernel scaffold; baseline (speedup 1.0000x reference)
import jax
import jax.numpy as jnp
from jax.experimental import pallas as pl


def kernel(input_ids, token_table, pos_table):
    raise NotImplementedError("write your pallas kernel here")


# speedup vs baseline: 1.5944x; 1.5944x over previous
import functools

import jax
import jax.numpy as jnp
from jax import lax
from jax.experimental import pallas as pl
from jax.experimental.pallas import tpu as pltpu
from jax.experimental.pallas import tpu_sc as plsc

B = 32
S = 1024
D = 768
L = 16
NC = 2
NS = 16
NW = NC * NS
SCHUNK = S // NW
NBUF = 4


def _body(ids_hbm, tok_hbm, pos_hbm, out_hbm,
          idx_v, pos_v, rows0, rows1, rows2, rows3,
          sg0, sg1, sg2, sg3, so0, so1, so2, so3):
    wid = lax.axis_index("s") * NC + lax.axis_index("c")
    s0 = pl.multiple_of(wid * SCHUNK, SCHUNK)

    bufs = (rows0, rows1, rows2, rows3)
    gsems = (sg0, sg1, sg2, sg3)
    osems = (so0, so1, so2, so3)

    for b in range(B):
        pltpu.make_async_copy(
            ids_hbm.at[pl.ds(b * S + SCHUNK * wid, SCHUNK)], idx_v.at[b], sg0).start()
    for b in range(B):
        pltpu.make_async_copy(
            ids_hbm.at[pl.ds(b * S + SCHUNK * wid, SCHUNK)], idx_v.at[b], sg0).wait()
    pltpu.sync_copy(pos_hbm.at[pl.ds(s0, SCHUNK)], pos_v)

    def gather_start(b, p):
        pltpu.make_async_copy(tok_hbm.at[idx_v.at[b]], bufs[p], gsems[p]).start()

    def gather_wait(b, p):
        pltpu.make_async_copy(tok_hbm.at[idx_v.at[b]], bufs[p], gsems[p]).wait()

    def out_start(b, p):
        pltpu.make_async_copy(bufs[p], out_hbm.at[b, pl.ds(s0, SCHUNK)], osems[p]).start()

    def out_wait(b, p):
        pltpu.make_async_copy(bufs[p], out_hbm.at[b, pl.ds(s0, SCHUNK)], osems[p]).wait()

    def add_pos(p):
        rows = bufs[p]

        def add_row(r, carry):
            for j in range(D // L):
                plsc.addupdate(rows.at[r, pl.ds(j * L, L)],
                               pos_v[r, pl.ds(j * L, L)])
            return carry

        lax.fori_loop(0, SCHUNK, add_row, 0)

    def half(b, k):
        if isinstance(b, int):
            if b + 2 < B:
                if b >= 2:
                    out_wait(b - 2, (b + 2) % NBUF)
                gather_start(b + 2, (b + 2) % NBUF)
        else:
            out_wait(b - 2, (k + 2) % NBUF)
            gather_start(b + 2, (k + 2) % NBUF)
        gather_wait(b, k)
        add_pos(k)
        out_start(b, k)

    gather_start(0, 0)
    gather_start(1, 1)
    for b in range(NBUF):
        half(b, b)

    def group(g, carry):
        b0 = g * NBUF
        for k in range(NBUF):
            half(b0 + k, k)
        return carry

    lax.fori_loop(1, B // NBUF - 1, group, 0)

    for b in range(B - NBUF, B):
        half(b, b % NBUF)
    for b in range(B - NBUF, B):
        out_wait(b, b % NBUF)


@jax.jit
def kernel(input_ids, token_table, pos_table):
    mesh = plsc.VectorSubcoreMesh(core_axis_name="c", subcore_axis_name="s")
    f = functools.partial(
        pl.kernel,
        mesh=mesh,
        out_type=jax.ShapeDtypeStruct((B, S, D), jnp.float32),
        scratch_types=[
            pltpu.VMEM((B, SCHUNK), jnp.int32),
            pltpu.VMEM((SCHUNK, D), jnp.float32),
            pltpu.VMEM((SCHUNK, D), jnp.float32),
            pltpu.VMEM((SCHUNK, D), jnp.float32),
            pltpu.VMEM((SCHUNK, D), jnp.float32),
            pltpu.VMEM((SCHUNK, D), jnp.float32),
            pltpu.SemaphoreType.DMA,
            pltpu.SemaphoreType.DMA,
            pltpu.SemaphoreType.DMA,
            pltpu.SemaphoreType.DMA,
            pltpu.SemaphoreType.DMA,
            pltpu.SemaphoreType.DMA,
            pltpu.SemaphoreType.DMA,
            pltpu.SemaphoreType.DMA,
        ],
    )(_body)
    return f(input_ids.astype(jnp.int32).reshape(-1), token_table, pos_table)
